# baseline (speedup 1.0000x reference)
import jax
import jax.numpy as jnp
from jax.experimental import pallas as pl


def _make_film_body(lanes):
    def _film_body(sel_ref, rows_ref, x_ref, out_ref):
        sel = sel_ref[...]
        g = jnp.dot(sel, rows_ref[:, :lanes], preferred_element_type=jnp.float32)
        b = jnp.dot(sel, rows_ref[:, lanes:], preferred_element_type=jnp.float32)
        out_ref[...] = x_ref[...] * g + b

    return _film_body


def kernel(feat, alt_idx, gamma, beta):
    bsz, seq, dim = feat.shape
    lanes = 128
    rows_per_batch = seq * dim // lanes
    total_r = bsz * rows_per_batch

    bb = 32
    blk_r = bb * rows_per_batch
    nb = bsz // bb

    x = feat.reshape(total_r, lanes)
    table = jnp.concatenate([gamma, gamma, beta, beta], axis=1)
    rows = jnp.take(table, alt_idx, axis=0)
    sel = (jnp.arange(blk_r)[:, None] // rows_per_batch
           == jnp.arange(bb)[None, :]).astype(jnp.float32)

    film = pl.pallas_call(
        _make_film_body(lanes),
        grid=(nb,),
        in_specs=[
            pl.BlockSpec((blk_r, bb), lambda i: (0, 0)),
            pl.BlockSpec((bb, 2 * lanes), lambda i: (i, 0)),
            pl.BlockSpec((blk_r, lanes), lambda i: (i, 0)),
        ],
        out_specs=pl.BlockSpec((blk_r, lanes), lambda i: (i, 0)),
        out_shape=jax.ShapeDtypeStruct((total_r, lanes), jnp.float32),
    )
    out = film(sel, rows, x)
    return out.reshape(bsz, seq, dim)
